# baseline (speedup 1.0000x reference)
import functools

import jax
import jax.numpy as jnp
from jax import lax
from jax.experimental import pallas as pl
from jax.experimental.pallas import tpu as pltpu
from jax.experimental.pallas import tpu_sc as plsc

N = 10000
D_IN = 128
H1 = 64
H2 = 32
E = 160000

NC = 2
NS = 16
NW = NC * NS
CH = 128
NCHUNK = 40
EPAD = NW * NCHUNK * CH
NPAD = 10240
RPT = NPAD // NS

_MESH = plsc.VectorSubcoreMesh(core_axis_name="c", subcore_axis_name="s")


def _deg_body(ed_hbm, out_hbm, ed_v, deg_v, semE0, semE1):
    cid = lax.axis_index("c")
    sid = lax.axis_index("s")
    wid = cid * NS + sid
    semE = (semE0, semE1)

    def ed_copy(c, b):
        return pltpu.make_async_copy(
            ed_hbm.at[wid * NCHUNK + c], ed_v.at[b], semE[b])

    ed_copy(0, 0).start()
    ed_copy(1, 1).start()

    seed16 = jnp.zeros((16,), jnp.float32) + jnp.where(
        wid == 0, jnp.float32(1.0), jnp.float32(0.0))

    def init_body(i, c):
        deg_v[pl.ds(i * 16, 16)] = seed16
        return c

    lax.fori_loop(0, N // 16, init_body, 0)

    def step_body(half, c0):
        for b in range(2):
            c = c0 + b
            ed_copy(c, b).wait()

            def vec_body(i, c2):
                idx = ed_v[b, 1, pl.ds(i * 16, 16)]
                val = plsc.bitcast(ed_v[b, 2, pl.ds(i * 16, 16)], jnp.float32)
                plsc.addupdate_scatter(deg_v, [idx], val)
                return c2

            lax.fori_loop(0, CH // 16, vec_body, 0)
            @pl.when(jnp.logical_not(c + 2 >= NCHUNK))
            def _():
                ed_copy(c + 2, b).start()
        return c0 + 2

    lax.fori_loop(0, NCHUNK // 2, step_body, 0)
    pltpu.sync_copy(deg_v, out_hbm.at[pl.ds(wid * N, N)])


_deg_call = pl.kernel(
    _deg_body,
    out_type=jax.ShapeDtypeStruct((NW * N,), jnp.float32),
    mesh=_MESH,
    compiler_params=pltpu.CompilerParams(
        needs_layout_passes=False, use_tc_tiling_on_sc=False),
    scratch_types=[
        pltpu.VMEM((2, 3, CH), jnp.int32),
        pltpu.VMEM((N,), jnp.float32),
        pltpu.SemaphoreType.DMA,
        pltpu.SemaphoreType.DMA,
    ],
)


def _mp_body(p_hbm, dinv_hbm, ed_hbm, zeros_hbm, out_hbm,
             ed_v, rows_v, dinv_v, self_v, acc_sh, semE0, semE1, semG0, semG1):
    cid = lax.axis_index("c")
    sid = lax.axis_index("s")
    wid = cid * NS + sid
    semE = (semE0, semE1)
    semG = (semG0, semG1)

    def ed_copy(c, b):
        return pltpu.make_async_copy(ed_hbm.at[wid * NCHUNK + c], ed_v.at[b],
                                     semE[b])

    def gather_copy(b):
        return pltpu.make_async_copy(p_hbm.at[ed_v.at[b, 0]], rows_v.at[b],
                                     semG[b])

    pltpu.sync_copy(dinv_hbm, dinv_v)

    def self_init(off, nrows):
        pltpu.sync_copy(p_hbm.at[pl.ds(off, nrows)], self_v.at[pl.ds(0, nrows)])

        def sgroup(i, c):
            dv = dinv_v[pl.ds(off + i * 16, 16)]
            n16 = dv * dv
            for k in range(16):
                w = n16[k]
                for q in range(H1 // 16):
                    self_v[i * 16 + k, pl.ds(q * 16, 16)] = (
                        self_v[i * 16 + k, pl.ds(q * 16, 16)] * w)
            return c

        lax.fori_loop(0, nrows // 16, sgroup, 0)
        pltpu.sync_copy(self_v.at[pl.ds(0, nrows)], acc_sh.at[pl.ds(off, nrows)])

    def zero_init(off, nrows):
        pltpu.sync_copy(zeros_hbm.at[pl.ds(0, nrows)], acc_sh.at[pl.ds(off, nrows)])

    @pl.when(cid == 0)
    def _():
        @pl.when(sid < NS - 1)
        def _():
            self_init(sid * RPT, RPT)
        @pl.when(sid == NS - 1)
        def _():
            self_init((NS - 1) * RPT, N - (NS - 1) * RPT)

    @pl.when(cid == 1)
    def _():
        @pl.when(sid < NS - 1)
        def _():
            zero_init(sid * RPT, RPT)
        @pl.when(sid == NS - 1)
        def _():
            zero_init((NS - 1) * RPT, N - (NS - 1) * RPT)

    plsc.subcore_barrier()

    ed_copy(0, 0).start()
    ed_copy(1, 1).start()
    ed_copy(0, 0).wait()
    gather_copy(0).start()

    def step_body(half, c0):
        for b in range(2):
            c = c0 + b
            last = c + 1 >= NCHUNK
            @pl.when(jnp.logical_not(last))
            def _():
                ed_copy(c + 1, 1 - b).wait()
                gather_copy(1 - b).start()
            gather_copy(b).wait()

            zero16 = jnp.zeros((16,), jnp.int32)
            rot16 = (lax.iota(jnp.int32, 16) + 1) % 16

            def group_body(i, carry):
                s16 = ed_v[b, 0, pl.ds(i * 16, 16)]
                d16 = ed_v[b, 1, pl.ds(i * 16, 16)]
                w16 = plsc.bitcast(ed_v[b, 2, pl.ds(i * 16, 16)], jnp.float32)
                a16 = plsc.load_gather(dinv_v, [s16])
                b16 = plsc.load_gather(dinv_v, [d16])
                n16 = a16 * b16 * w16
                for k in range(16):
                    wv = jnp.take_along_axis(n16, zero16, axis=0)
                    if k < 15:
                        n16 = jnp.take_along_axis(n16, rot16, axis=0)
                    for q in range(H1 // 16):
                        e = i * 16 + k
                        rows_v[b, e, pl.ds(q * 16, 16)] = (
                            rows_v[b, e, pl.ds(q * 16, 16)] * wv)
                return carry

            lax.fori_loop(0, CH // 16, group_body, 0)
            pltpu.sync_copy(rows_v.at[b], acc_sh.at[ed_v.at[b, 1]], add=True)
            @pl.when(jnp.logical_not(c + 2 >= NCHUNK))
            def _():
                ed_copy(c + 2, b).start()
        return c0 + 2

    lax.fori_loop(0, NCHUNK // 2, step_body, 0)
    plsc.subcore_barrier()

    @pl.when(sid < NS - 1)
    def _():
        pltpu.sync_copy(acc_sh.at[pl.ds(sid * RPT, RPT)],
                        out_hbm.at[cid, pl.ds(sid * RPT, RPT)])

    @pl.when(sid == NS - 1)
    def _():
        pltpu.sync_copy(acc_sh.at[pl.ds((NS - 1) * RPT, N - (NS - 1) * RPT)],
                        out_hbm.at[cid, pl.ds((NS - 1) * RPT, N - (NS - 1) * RPT)])


_mp_call = pl.kernel(
    _mp_body,
    out_type=jax.ShapeDtypeStruct((NC, NPAD, H1), jnp.float32),
    mesh=_MESH,
    compiler_params=pltpu.CompilerParams(
        needs_layout_passes=False, use_tc_tiling_on_sc=False),
    scratch_types=[
        pltpu.VMEM((2, 3, CH), jnp.int32),
        pltpu.VMEM((2, CH, H1), jnp.float32),
        pltpu.VMEM((N,), jnp.float32),
        pltpu.VMEM((RPT, H1), jnp.float32),
        pltpu.VMEM_SHARED((NPAD, H1), jnp.float32),
        pltpu.SemaphoreType.DMA,
        pltpu.SemaphoreType.DMA,
        pltpu.SemaphoreType.DMA,
        pltpu.SemaphoreType.DMA,
    ],
)


def _dinv_body(degp_ref, o_ref):
    o_ref[...] = lax.rsqrt(jnp.sum(degp_ref[...], axis=0, keepdims=True))


def _dinv_call(degp):
    BM = 2048
    return pl.pallas_call(
        _dinv_body,
        grid=(pl.cdiv(N, BM),),
        in_specs=[pl.BlockSpec((NW, BM), lambda i: (0, i))],
        out_specs=pl.BlockSpec((1, BM), lambda i: (0, i)),
        out_shape=jax.ShapeDtypeStruct((1, N), jnp.float32),
    )(degp)


def _xw_body(x_ref, w_ref, o_ref):
    o_ref[...] = jax.lax.dot_general(
        x_ref[...], w_ref[...], (((1,), (0,)), ((), ())),
        preferred_element_type=jnp.float32)


def _xw_call(x, W1):
    BM = 2000
    return pl.pallas_call(
        _xw_body,
        grid=(N // BM,),
        in_specs=[pl.BlockSpec((BM, D_IN), lambda i: (i, 0)),
                  pl.BlockSpec((D_IN, H1), lambda i: (0, 0))],
        out_specs=pl.BlockSpec((BM, H1), lambda i: (i, 0)),
        out_shape=jax.ShapeDtypeStruct((N, H1), jnp.float32),
    )(x, W1)


def _hw_body(s_ref, b_ref, w_ref, o_ref):
    h = jax.nn.relu(s_ref[0] + s_ref[1] + b_ref[...])
    o_ref[...] = jax.lax.dot_general(
        h, w_ref[...], (((1,), (0,)), ((), ())),
        preferred_element_type=jnp.float32)


def _hw_call(S1, b1, Wc):
    BM = 2000
    return pl.pallas_call(
        _hw_body,
        grid=(N // BM,),
        in_specs=[pl.BlockSpec((NC, BM, H1), lambda i: (0, i, 0)),
                  pl.BlockSpec((1, H1), lambda i: (0, 0)),
                  pl.BlockSpec((H1, H1), lambda i: (0, 0))],
        out_specs=pl.BlockSpec((BM, H1), lambda i: (i, 0)),
        out_shape=jax.ShapeDtypeStruct((N, H1), jnp.float32),
    )(S1, b1, Wc)


def _mulv_body(s_ref, b_ref, mu_ref, lv_ref):
    o = s_ref[0] + s_ref[1] + b_ref[...]
    mu_ref[...] = o[:, :H2]
    lv_ref[...] = o[:, H2:]


def _mulv_call(S2, bc):
    BM = 2000
    return pl.pallas_call(
        _mulv_body,
        grid=(N // BM,),
        in_specs=[pl.BlockSpec((NC, BM, H1), lambda i: (0, i, 0)),
                  pl.BlockSpec((1, H1), lambda i: (0, 0))],
        out_specs=[pl.BlockSpec((BM, H2), lambda i: (i, 0)),
                   pl.BlockSpec((BM, H2), lambda i: (i, 0))],
        out_shape=[jax.ShapeDtypeStruct((N, H2), jnp.float32),
                   jax.ShapeDtypeStruct((N, H2), jnp.float32)],
    )(S2, bc)


def _zzt_body(zl_ref, zr_ref, o_ref):
    o_ref[...] = jax.lax.dot_general(
        zl_ref[...], zr_ref[...], (((1,), (1,)), ((), ())),
        preferred_element_type=jnp.float32)


def _zzt(z):
    BM, BN = 512, 4992
    return pl.pallas_call(
        _zzt_body,
        grid=(pl.cdiv(N, BM), pl.cdiv(N, BN)),
        in_specs=[pl.BlockSpec((BM, H2), lambda i, j: (i, 0)),
                  pl.BlockSpec((BN, H2), lambda i, j: (j, 0))],
        out_specs=pl.BlockSpec((BM, BN), lambda i, j: (i, j)),
        out_shape=jax.ShapeDtypeStruct((N, N), jnp.float32),
    )(z, z)


def kernel(x, adj, edge_weight, W1, b1, W2, b2, W3, b3):
    src = adj[0].astype(jnp.int32)
    dst = adj[1].astype(jnp.int32)
    npad = EPAD - E
    padi = (jnp.arange(npad, dtype=jnp.int32) * 7) % N
    padf = jnp.zeros((npad,), jnp.float32)
    src_f = jnp.concatenate([src, padi])
    dst_f = jnp.concatenate([dst, padi])
    ew_f = jnp.concatenate([edge_weight, padf])
    zeros = jnp.zeros((RPT, H1), jnp.float32)

    tot = NW * NCHUNK
    ed = jnp.stack([src_f.reshape(tot, CH), dst_f.reshape(tot, CH),
                    lax.bitcast_convert_type(ew_f, jnp.int32).reshape(tot, CH)],
                   axis=1)
    degp = _deg_call(ed).reshape(NW, N)
    dinv = _dinv_call(degp).reshape(N)
    P1 = _xw_call(x, W1)
    S1 = _mp_call(P1, dinv, ed, zeros)
    Wc = jnp.concatenate([W2, W3], axis=1)
    bc = jnp.concatenate([b2, b3]).reshape(1, H1)
    P2 = _hw_call(S1, b1.reshape(1, H1), Wc)
    S2 = _mp_call(P2, dinv, ed, zeros)
    mu, logvar = _mulv_call(S2, bc)
    adj_rec = _zzt(mu)
    return (adj_rec, mu, logvar)


# speedup vs baseline: 1.1607x; 1.1607x over previous
import functools

import jax
import jax.numpy as jnp
from jax import lax
from jax.experimental import pallas as pl
from jax.experimental.pallas import tpu as pltpu
from jax.experimental.pallas import tpu_sc as plsc

N = 10000
D_IN = 128
H1 = 64
H2 = 32
E = 160000

NC = 2
NS = 16
NW = NC * NS
CH = 128
NCHUNK = 40
EPAD = NW * NCHUNK * CH
NPAD = 10240
RPT = NPAD // NS

_MESH = plsc.VectorSubcoreMesh(core_axis_name="c", subcore_axis_name="s")


def _deg_body(ed_hbm, out_hbm, ed_v, deg_v, semE0, semE1):
    cid = lax.axis_index("c")
    sid = lax.axis_index("s")
    wid = cid * NS + sid
    semE = (semE0, semE1)

    def ed_copy(c, b):
        return pltpu.make_async_copy(
            ed_hbm.at[wid * NCHUNK + c], ed_v.at[b], semE[b])

    ed_copy(0, 0).start()
    ed_copy(1, 1).start()

    seed16 = jnp.zeros((16,), jnp.float32) + jnp.where(
        wid == 0, jnp.float32(1.0), jnp.float32(0.0))

    def init_body(i, c):
        deg_v[pl.ds(i * 16, 16)] = seed16
        return c

    lax.fori_loop(0, N // 16, init_body, 0)

    def step_body(half, c0):
        for b in range(2):
            c = c0 + b
            ed_copy(c, b).wait()

            def vec_body(i, c2):
                idx = ed_v[b, 1, pl.ds(i * 16, 16)]
                val = plsc.bitcast(ed_v[b, 2, pl.ds(i * 16, 16)], jnp.float32)
                plsc.addupdate_scatter(deg_v, [idx], val)
                return c2

            lax.fori_loop(0, CH // 16, vec_body, 0)
            @pl.when(jnp.logical_not(c + 2 >= NCHUNK))
            def _():
                ed_copy(c + 2, b).start()
        return c0 + 2

    lax.fori_loop(0, NCHUNK // 2, step_body, 0)
    pltpu.sync_copy(deg_v, out_hbm.at[pl.ds(wid * N, N)])


_deg_call = pl.kernel(
    _deg_body,
    out_type=jax.ShapeDtypeStruct((NW * N,), jnp.float32),
    mesh=_MESH,
    compiler_params=pltpu.CompilerParams(
        needs_layout_passes=False, use_tc_tiling_on_sc=False),
    scratch_types=[
        pltpu.VMEM((2, 3, CH), jnp.int32),
        pltpu.VMEM((N,), jnp.float32),
        pltpu.SemaphoreType.DMA,
        pltpu.SemaphoreType.DMA,
    ],
)


def _mp_body(p_hbm, dinv_hbm, ed_hbm, zeros_hbm, out_hbm,
             ed_v, rows_v, dinv_v, self_v, acc_sh, semE0, semE1, semG0, semG1):
    cid = lax.axis_index("c")
    sid = lax.axis_index("s")
    wid = cid * NS + sid
    semE = (semE0, semE1)
    semG = (semG0, semG1)

    def ed_copy(c, b):
        return pltpu.make_async_copy(ed_hbm.at[wid * NCHUNK + c], ed_v.at[b],
                                     semE[b])

    def gather_copy(b):
        return pltpu.make_async_copy(p_hbm.at[ed_v.at[b, 0]], rows_v.at[b],
                                     semG[b])

    pltpu.sync_copy(dinv_hbm, dinv_v)

    def self_init(off, nrows):
        pltpu.sync_copy(p_hbm.at[pl.ds(off, nrows)], self_v.at[pl.ds(0, nrows)])

        def sgroup(i, c):
            dv = dinv_v[pl.ds(off + i * 16, 16)]
            n16 = dv * dv
            for k in range(16):
                w = n16[k]
                for q in range(H1 // 16):
                    self_v[i * 16 + k, pl.ds(q * 16, 16)] = (
                        self_v[i * 16 + k, pl.ds(q * 16, 16)] * w)
            return c

        lax.fori_loop(0, nrows // 16, sgroup, 0)
        pltpu.sync_copy(self_v.at[pl.ds(0, nrows)], acc_sh.at[pl.ds(off, nrows)])

    def zero_init(off, nrows):
        pltpu.sync_copy(zeros_hbm.at[pl.ds(0, nrows)], acc_sh.at[pl.ds(off, nrows)])

    @pl.when(cid == 0)
    def _():
        @pl.when(sid < NS - 1)
        def _():
            self_init(sid * RPT, RPT)
        @pl.when(sid == NS - 1)
        def _():
            self_init((NS - 1) * RPT, N - (NS - 1) * RPT)

    @pl.when(cid == 1)
    def _():
        @pl.when(sid < NS - 1)
        def _():
            zero_init(sid * RPT, RPT)
        @pl.when(sid == NS - 1)
        def _():
            zero_init((NS - 1) * RPT, N - (NS - 1) * RPT)

    plsc.subcore_barrier()

    ed_copy(0, 0).start()
    ed_copy(1, 1).start()
    ed_copy(0, 0).wait()
    gather_copy(0).start()

    def step_body(half, c0):
        for b in range(2):
            c = c0 + b
            last = c + 1 >= NCHUNK
            @pl.when(jnp.logical_not(last))
            def _():
                ed_copy(c + 1, 1 - b).wait()
                gather_copy(1 - b).start()
            gather_copy(b).wait()

            zero16 = jnp.zeros((16,), jnp.int32)
            rot16 = (lax.iota(jnp.int32, 16) + 1) % 16

            def group_body(i, carry):
                s16 = ed_v[b, 0, pl.ds(i * 16, 16)]
                d16 = ed_v[b, 1, pl.ds(i * 16, 16)]
                w16 = plsc.bitcast(ed_v[b, 2, pl.ds(i * 16, 16)], jnp.float32)
                a16 = plsc.load_gather(dinv_v, [s16])
                b16 = plsc.load_gather(dinv_v, [d16])
                n16 = a16 * b16 * w16
                for k in range(16):
                    wv = jnp.take_along_axis(n16, zero16, axis=0)
                    if k < 15:
                        n16 = jnp.take_along_axis(n16, rot16, axis=0)
                    for q in range(H1 // 16):
                        e = i * 16 + k
                        rows_v[b, e, pl.ds(q * 16, 16)] = (
                            rows_v[b, e, pl.ds(q * 16, 16)] * wv)
                return carry

            lax.fori_loop(0, CH // 16, group_body, 0)
            pltpu.sync_copy(rows_v.at[b], acc_sh.at[ed_v.at[b, 1]], add=True)
            @pl.when(jnp.logical_not(c + 2 >= NCHUNK))
            def _():
                ed_copy(c + 2, b).start()
        return c0 + 2

    lax.fori_loop(0, NCHUNK // 2, step_body, 0)
    plsc.subcore_barrier()

    @pl.when(sid < NS - 1)
    def _():
        pltpu.sync_copy(acc_sh.at[pl.ds(sid * RPT, RPT)],
                        out_hbm.at[cid, pl.ds(sid * RPT, RPT)])

    @pl.when(sid == NS - 1)
    def _():
        pltpu.sync_copy(acc_sh.at[pl.ds((NS - 1) * RPT, N - (NS - 1) * RPT)],
                        out_hbm.at[cid, pl.ds((NS - 1) * RPT, N - (NS - 1) * RPT)])


_mp_call = pl.kernel(
    _mp_body,
    out_type=jax.ShapeDtypeStruct((NC, NPAD, H1), jnp.float32),
    mesh=_MESH,
    compiler_params=pltpu.CompilerParams(
        needs_layout_passes=False, use_tc_tiling_on_sc=False),
    scratch_types=[
        pltpu.VMEM((2, 3, CH), jnp.int32),
        pltpu.VMEM((2, CH, H1), jnp.float32),
        pltpu.VMEM((N,), jnp.float32),
        pltpu.VMEM((RPT, H1), jnp.float32),
        pltpu.VMEM_SHARED((NPAD, H1), jnp.float32),
        pltpu.SemaphoreType.DMA,
        pltpu.SemaphoreType.DMA,
        pltpu.SemaphoreType.DMA,
        pltpu.SemaphoreType.DMA,
    ],
)


def _dinv_body(degp_ref, o_ref):
    o_ref[...] = lax.rsqrt(jnp.sum(degp_ref[...], axis=0, keepdims=True))


def _dinv_call(degp):
    BM = 2048
    return pl.pallas_call(
        _dinv_body,
        grid=(pl.cdiv(N, BM),),
        in_specs=[pl.BlockSpec((NW, BM), lambda i: (0, i))],
        out_specs=pl.BlockSpec((1, BM), lambda i: (0, i)),
        out_shape=jax.ShapeDtypeStruct((1, N), jnp.float32),
    )(degp)


def _xw_body(x_ref, w_ref, o_ref):
    o_ref[...] = jax.lax.dot_general(
        x_ref[...], w_ref[...], (((1,), (0,)), ((), ())),
        preferred_element_type=jnp.float32)


def _xw_call(x, W1):
    BM = 2000
    return pl.pallas_call(
        _xw_body,
        grid=(N // BM,),
        in_specs=[pl.BlockSpec((BM, D_IN), lambda i: (i, 0)),
                  pl.BlockSpec((D_IN, H1), lambda i: (0, 0))],
        out_specs=pl.BlockSpec((BM, H1), lambda i: (i, 0)),
        out_shape=jax.ShapeDtypeStruct((N, H1), jnp.float32),
    )(x, W1)


def _hw_body(s_ref, b_ref, w_ref, o_ref):
    h = jax.nn.relu(s_ref[0] + s_ref[1] + b_ref[...])
    o_ref[...] = jax.lax.dot_general(
        h, w_ref[...], (((1,), (0,)), ((), ())),
        preferred_element_type=jnp.float32)


def _hw_call(S1, b1, Wc):
    BM = 2000
    return pl.pallas_call(
        _hw_body,
        grid=(N // BM,),
        in_specs=[pl.BlockSpec((NC, BM, H1), lambda i: (0, i, 0)),
                  pl.BlockSpec((1, H1), lambda i: (0, 0)),
                  pl.BlockSpec((H1, H1), lambda i: (0, 0))],
        out_specs=pl.BlockSpec((BM, H1), lambda i: (i, 0)),
        out_shape=jax.ShapeDtypeStruct((N, H1), jnp.float32),
    )(S1, b1, Wc)


def _mulv_body(s_ref, b_ref, mu_ref, lv_ref):
    o = s_ref[0] + s_ref[1] + b_ref[...]
    mu_ref[...] = o[:, :H2]
    lv_ref[...] = o[:, H2:]


def _mulv_call(S2, bc):
    BM = 2000
    return pl.pallas_call(
        _mulv_body,
        grid=(N // BM,),
        in_specs=[pl.BlockSpec((NC, BM, H1), lambda i: (0, i, 0)),
                  pl.BlockSpec((1, H1), lambda i: (0, 0))],
        out_specs=[pl.BlockSpec((BM, H2), lambda i: (i, 0)),
                   pl.BlockSpec((BM, H2), lambda i: (i, 0))],
        out_shape=[jax.ShapeDtypeStruct((N, H2), jnp.float32),
                   jax.ShapeDtypeStruct((N, H2), jnp.float32)],
    )(S2, bc)


def _zzt_body(zl_ref, zr_ref, o_ref):
    o_ref[...] = jax.lax.dot_general(
        zl_ref[...], zr_ref[...], (((1,), (1,)), ((), ())),
        preferred_element_type=jnp.float32)


def _zzt(z):
    BM, BN = 2048, 2048
    return pl.pallas_call(
        _zzt_body,
        grid=(pl.cdiv(N, BM), pl.cdiv(N, BN)),
        in_specs=[pl.BlockSpec((BM, H2), lambda i, j: (i, 0)),
                  pl.BlockSpec((BN, H2), lambda i, j: (j, 0))],
        out_specs=pl.BlockSpec((BM, BN), lambda i, j: (i, j)),
        out_shape=jax.ShapeDtypeStruct((N, N), jnp.float32),
    )(z, z)


def kernel(x, adj, edge_weight, W1, b1, W2, b2, W3, b3):
    src = adj[0].astype(jnp.int32)
    dst = adj[1].astype(jnp.int32)
    npad = EPAD - E
    padi = (jnp.arange(npad, dtype=jnp.int32) * 7) % N
    padf = jnp.zeros((npad,), jnp.float32)
    src_f = jnp.concatenate([src, padi])
    dst_f = jnp.concatenate([dst, padi])
    ew_f = jnp.concatenate([edge_weight, padf])
    zeros = jnp.zeros((RPT, H1), jnp.float32)

    tot = NW * NCHUNK
    ed = jnp.stack([src_f.reshape(tot, CH), dst_f.reshape(tot, CH),
                    lax.bitcast_convert_type(ew_f, jnp.int32).reshape(tot, CH)],
                   axis=1)
    degp = _deg_call(ed).reshape(NW, N)
    dinv = _dinv_call(degp).reshape(N)
    P1 = _xw_call(x, W1)
    S1 = _mp_call(P1, dinv, ed, zeros)
    Wc = jnp.concatenate([W2, W3], axis=1)
    bc = jnp.concatenate([b2, b3]).reshape(1, H1)
    P2 = _hw_call(S1, b1.reshape(1, H1), Wc)
    S2 = _mp_call(P2, dinv, ed, zeros)
    mu, logvar = _mulv_call(S2, bc)
    adj_rec = _zzt(mu)
    return (adj_rec, mu, logvar)


# speedup vs baseline: 1.1667x; 1.0052x over previous
import functools

import jax
import jax.numpy as jnp
from jax import lax
from jax.experimental import pallas as pl
from jax.experimental.pallas import tpu as pltpu
from jax.experimental.pallas import tpu_sc as plsc

N = 10000
D_IN = 128
H1 = 64
H2 = 32
E = 160000

NC = 2
NS = 16
NW = NC * NS
CH = 128
NCHUNK = 40
EPAD = NW * NCHUNK * CH
NPAD = 10240
RPT = NPAD // NS

_MESH = plsc.VectorSubcoreMesh(core_axis_name="c", subcore_axis_name="s")


def _deg_body(ed_hbm, out_hbm, ed_v, deg_v, semE0, semE1):
    cid = lax.axis_index("c")
    sid = lax.axis_index("s")
    wid = cid * NS + sid
    semE = (semE0, semE1)

    def ed_copy(c, b):
        return pltpu.make_async_copy(
            ed_hbm.at[wid * NCHUNK + c], ed_v.at[b], semE[b])

    ed_copy(0, 0).start()
    ed_copy(1, 1).start()

    seed16 = jnp.zeros((16,), jnp.float32) + jnp.where(
        wid == 0, jnp.float32(1.0), jnp.float32(0.0))

    def init_body(i, c):
        deg_v[pl.ds(i * 16, 16)] = seed16
        return c

    lax.fori_loop(0, N // 16, init_body, 0)

    def step_body(half, c0):
        for b in range(2):
            c = c0 + b
            ed_copy(c, b).wait()

            def vec_body(i, c2):
                idx = ed_v[b, 1, pl.ds(i * 16, 16)]
                val = plsc.bitcast(ed_v[b, 2, pl.ds(i * 16, 16)], jnp.float32)
                plsc.addupdate_scatter(deg_v, [idx], val)
                return c2

            lax.fori_loop(0, CH // 16, vec_body, 0)
            @pl.when(jnp.logical_not(c + 2 >= NCHUNK))
            def _():
                ed_copy(c + 2, b).start()
        return c0 + 2

    lax.fori_loop(0, NCHUNK // 2, step_body, 0)
    pltpu.sync_copy(deg_v, out_hbm.at[pl.ds(wid * N, N)])


_deg_call = pl.kernel(
    _deg_body,
    out_type=jax.ShapeDtypeStruct((NW * N,), jnp.float32),
    mesh=_MESH,
    compiler_params=pltpu.CompilerParams(
        needs_layout_passes=False, use_tc_tiling_on_sc=False),
    scratch_types=[
        pltpu.VMEM((2, 3, CH), jnp.int32),
        pltpu.VMEM((N,), jnp.float32),
        pltpu.SemaphoreType.DMA,
        pltpu.SemaphoreType.DMA,
    ],
)


def _mp_body(p_hbm, dinv_hbm, ed_hbm, zeros_hbm, out_hbm,
             ed_v, rows_v, dinv_v, self_v, acc_sh, semE0, semE1, semG0, semG1):
    cid = lax.axis_index("c")
    sid = lax.axis_index("s")
    wid = cid * NS + sid
    semE = (semE0, semE1)
    semG = (semG0, semG1)

    def ed_copy(c, b):
        return pltpu.make_async_copy(ed_hbm.at[wid * NCHUNK + c], ed_v.at[b],
                                     semE[b])

    def gather_copy(b):
        return pltpu.make_async_copy(p_hbm.at[ed_v.at[b, 0]], rows_v.at[b],
                                     semG[b])

    pltpu.sync_copy(dinv_hbm, dinv_v)

    def self_init(off, nrows):
        pltpu.sync_copy(p_hbm.at[pl.ds(off, nrows)], self_v.at[pl.ds(0, nrows)])

        def sgroup(i, c):
            dv = dinv_v[pl.ds(off + i * 16, 16)]
            n16 = dv * dv
            for k in range(16):
                w = n16[k]
                for q in range(H1 // 16):
                    self_v[i * 16 + k, pl.ds(q * 16, 16)] = (
                        self_v[i * 16 + k, pl.ds(q * 16, 16)] * w)
            return c

        lax.fori_loop(0, nrows // 16, sgroup, 0)
        pltpu.sync_copy(self_v.at[pl.ds(0, nrows)], acc_sh.at[pl.ds(off, nrows)])

    def zero_init(off, nrows):
        pltpu.sync_copy(zeros_hbm.at[pl.ds(0, nrows)], acc_sh.at[pl.ds(off, nrows)])

    @pl.when(cid == 0)
    def _():
        @pl.when(sid < NS - 1)
        def _():
            self_init(sid * RPT, RPT)
        @pl.when(sid == NS - 1)
        def _():
            self_init((NS - 1) * RPT, N - (NS - 1) * RPT)

    @pl.when(cid == 1)
    def _():
        @pl.when(sid < NS - 1)
        def _():
            zero_init(sid * RPT, RPT)
        @pl.when(sid == NS - 1)
        def _():
            zero_init((NS - 1) * RPT, N - (NS - 1) * RPT)

    plsc.subcore_barrier()

    ed_copy(0, 0).start()
    ed_copy(1, 1).start()
    ed_copy(0, 0).wait()
    gather_copy(0).start()

    def step_body(half, c0):
        for b in range(2):
            c = c0 + b
            last = c + 1 >= NCHUNK
            @pl.when(jnp.logical_not(last))
            def _():
                ed_copy(c + 1, 1 - b).wait()
                gather_copy(1 - b).start()
            gather_copy(b).wait()

            zero16 = jnp.zeros((16,), jnp.int32)
            rot16 = (lax.iota(jnp.int32, 16) + 1) % 16

            def group_body(i, carry):
                s16 = ed_v[b, 0, pl.ds(i * 16, 16)]
                d16 = ed_v[b, 1, pl.ds(i * 16, 16)]
                w16 = plsc.bitcast(ed_v[b, 2, pl.ds(i * 16, 16)], jnp.float32)
                a16 = plsc.load_gather(dinv_v, [s16])
                b16 = plsc.load_gather(dinv_v, [d16])
                n16 = a16 * b16 * w16
                for k in range(16):
                    wv = jnp.take_along_axis(n16, zero16, axis=0)
                    if k < 15:
                        n16 = jnp.take_along_axis(n16, rot16, axis=0)
                    for q in range(H1 // 16):
                        e = i * 16 + k
                        rows_v[b, e, pl.ds(q * 16, 16)] = (
                            rows_v[b, e, pl.ds(q * 16, 16)] * wv)
                return carry

            lax.fori_loop(0, CH // 16, group_body, 0)
            pltpu.sync_copy(rows_v.at[b], acc_sh.at[ed_v.at[b, 1]], add=True)
            @pl.when(jnp.logical_not(c + 2 >= NCHUNK))
            def _():
                ed_copy(c + 2, b).start()
        return c0 + 2

    lax.fori_loop(0, NCHUNK // 2, step_body, 0)
    plsc.subcore_barrier()

    @pl.when(sid < NS - 1)
    def _():
        pltpu.sync_copy(acc_sh.at[pl.ds(sid * RPT, RPT)],
                        out_hbm.at[cid, pl.ds(sid * RPT, RPT)])

    @pl.when(sid == NS - 1)
    def _():
        pltpu.sync_copy(acc_sh.at[pl.ds((NS - 1) * RPT, N - (NS - 1) * RPT)],
                        out_hbm.at[cid, pl.ds((NS - 1) * RPT, N - (NS - 1) * RPT)])


_mp_call = pl.kernel(
    _mp_body,
    out_type=jax.ShapeDtypeStruct((NC, NPAD, H1), jnp.float32),
    mesh=_MESH,
    compiler_params=pltpu.CompilerParams(
        needs_layout_passes=False, use_tc_tiling_on_sc=False),
    scratch_types=[
        pltpu.VMEM((2, 3, CH), jnp.int32),
        pltpu.VMEM((2, CH, H1), jnp.float32),
        pltpu.VMEM((N,), jnp.float32),
        pltpu.VMEM((RPT, H1), jnp.float32),
        pltpu.VMEM_SHARED((NPAD, H1), jnp.float32),
        pltpu.SemaphoreType.DMA,
        pltpu.SemaphoreType.DMA,
        pltpu.SemaphoreType.DMA,
        pltpu.SemaphoreType.DMA,
    ],
)


def _dinv_body(degp_ref, o_ref):
    o_ref[...] = lax.rsqrt(jnp.sum(degp_ref[...], axis=0, keepdims=True))


def _dinv_call(degp):
    BM = 2048
    return pl.pallas_call(
        _dinv_body,
        grid=(pl.cdiv(N, BM),),
        in_specs=[pl.BlockSpec((NW, BM), lambda i: (0, i))],
        out_specs=pl.BlockSpec((1, BM), lambda i: (0, i)),
        out_shape=jax.ShapeDtypeStruct((1, N), jnp.float32),
    )(degp)


def _xw_body(x_ref, w_ref, o_ref):
    o_ref[...] = jax.lax.dot_general(
        x_ref[...], w_ref[...], (((1,), (0,)), ((), ())),
        preferred_element_type=jnp.float32)


def _xw_call(x, W1):
    BM = 2000
    return pl.pallas_call(
        _xw_body,
        grid=(N // BM,),
        in_specs=[pl.BlockSpec((BM, D_IN), lambda i: (i, 0)),
                  pl.BlockSpec((D_IN, H1), lambda i: (0, 0))],
        out_specs=pl.BlockSpec((BM, H1), lambda i: (i, 0)),
        out_shape=jax.ShapeDtypeStruct((N, H1), jnp.float32),
    )(x, W1)


def _hw_body(s_ref, b_ref, w_ref, o_ref):
    h = jax.nn.relu(s_ref[0] + s_ref[1] + b_ref[...])
    o_ref[...] = jax.lax.dot_general(
        h, w_ref[...], (((1,), (0,)), ((), ())),
        preferred_element_type=jnp.float32)


def _hw_call(S1, b1, Wc):
    BM = 2000
    return pl.pallas_call(
        _hw_body,
        grid=(N // BM,),
        in_specs=[pl.BlockSpec((NC, BM, H1), lambda i: (0, i, 0)),
                  pl.BlockSpec((1, H1), lambda i: (0, 0)),
                  pl.BlockSpec((H1, H1), lambda i: (0, 0))],
        out_specs=pl.BlockSpec((BM, H1), lambda i: (i, 0)),
        out_shape=jax.ShapeDtypeStruct((N, H1), jnp.float32),
    )(S1, b1, Wc)


def _mulv_body(s_ref, b_ref, mu_ref, lv_ref):
    o = s_ref[0] + s_ref[1] + b_ref[...]
    mu_ref[...] = o[:, :H2]
    lv_ref[...] = o[:, H2:]


def _mulv_call(S2, bc):
    BM = 2000
    return pl.pallas_call(
        _mulv_body,
        grid=(N // BM,),
        in_specs=[pl.BlockSpec((NC, BM, H1), lambda i: (0, i, 0)),
                  pl.BlockSpec((1, H1), lambda i: (0, 0))],
        out_specs=[pl.BlockSpec((BM, H2), lambda i: (i, 0)),
                   pl.BlockSpec((BM, H2), lambda i: (i, 0))],
        out_shape=[jax.ShapeDtypeStruct((N, H2), jnp.float32),
                   jax.ShapeDtypeStruct((N, H2), jnp.float32)],
    )(S2, bc)


def _zzt_body(zl_ref, zr_ref, o_ref):
    o_ref[...] = jax.lax.dot_general(
        zl_ref[...], zr_ref[...], (((1,), (1,)), ((), ())),
        preferred_element_type=jnp.float32)


def _zzt(z):
    BM, BN = 2560, 2560
    return pl.pallas_call(
        _zzt_body,
        grid=(pl.cdiv(N, BM), pl.cdiv(N, BN)),
        in_specs=[pl.BlockSpec((BM, H2), lambda i, j: (i, 0)),
                  pl.BlockSpec((BN, H2), lambda i, j: (j, 0))],
        out_specs=pl.BlockSpec((BM, BN), lambda i, j: (i, j)),
        out_shape=jax.ShapeDtypeStruct((N, N), jnp.float32),
    )(z, z)


def kernel(x, adj, edge_weight, W1, b1, W2, b2, W3, b3):
    src = adj[0].astype(jnp.int32)
    dst = adj[1].astype(jnp.int32)
    npad = EPAD - E
    padi = (jnp.arange(npad, dtype=jnp.int32) * 7) % N
    padf = jnp.zeros((npad,), jnp.float32)
    src_f = jnp.concatenate([src, padi])
    dst_f = jnp.concatenate([dst, padi])
    ew_f = jnp.concatenate([edge_weight, padf])
    zeros = jnp.zeros((RPT, H1), jnp.float32)

    tot = NW * NCHUNK
    ed = jnp.stack([src_f.reshape(tot, CH), dst_f.reshape(tot, CH),
                    lax.bitcast_convert_type(ew_f, jnp.int32).reshape(tot, CH)],
                   axis=1)
    degp = _deg_call(ed).reshape(NW, N)
    dinv = _dinv_call(degp).reshape(N)
    P1 = _xw_call(x, W1)
    S1 = _mp_call(P1, dinv, ed, zeros)
    Wc = jnp.concatenate([W2, W3], axis=1)
    bc = jnp.concatenate([b2, b3]).reshape(1, H1)
    P2 = _hw_call(S1, b1.reshape(1, H1), Wc)
    S2 = _mp_call(P2, dinv, ed, zeros)
    mu, logvar = _mulv_call(S2, bc)
    adj_rec = _zzt(mu)
    return (adj_rec, mu, logvar)
